# baseline (speedup 1.0000x reference)
import functools

import jax
import jax.numpy as jnp
from jax import lax
from jax.experimental import pallas as pl
from jax.experimental.pallas import tpu as pltpu
from jax.experimental.pallas import tpu_sc as plsc


def _top16_sorted(d2_vregs):
    runs = [lax.sort(v) for v in d2_vregs]
    while len(runs) > 1:
        nxt = []
        for a, b in zip(runs[0::2], runs[1::2]):
            nxt.append(lax.sort(jnp.minimum(a, lax.rev(b, (0,)))))
        runs = nxt
    return runs[0]


def _sc_stage_rows(nrows, nsv, inv2bw2, sxv, syv, szv, dxv, dyv, dzv,
                   d2buf, obuf):
    ns = nsv * 16

    def row_body(r, carry):
        cb = (r // 16) * 16
        off = r - cb
        lane = lax.iota(jnp.int32, 16)

        def splat(ref):
            v = ref[pl.ds(cb, 16)]
            return jnp.full((16,), jnp.sum(jnp.where(lane == off, v, 0.0)))

        dnx = splat(dxv)
        dny = splat(dyv)
        dnz = splat(dzv)
        d2s = []
        for j in range(nsv):
            sl = pl.ds(j * 16, 16)
            ax = sxv[sl] - dnx
            ay = syv[sl] - dny
            az = szv[sl] - dnz
            d2j = ax * ax + ay * ay + az * az
            d2buf[sl] = d2j
            d2s.append(d2j)
        t16 = _top16_sorted(d2s)
        tv = jnp.full((16,), jnp.max(t16))
        acc = jnp.zeros((16,), jnp.float32)
        for j in range(nsv):
            sl = pl.ds(j * 16, 16)
            wj = jnp.where(d2buf[sl] <= tv, jnp.exp(d2buf[sl] * (-inv2bw2)), 0.0)
            d2buf[sl] = wj
            acc = acc + wj
        denom = jnp.full((16,), jnp.sum(acc)) + 1e-8
        invv = jnp.ones((16,), jnp.float32) / denom
        for j in range(nsv):
            obuf[pl.ds(r * ns + j * 16, 16)] = d2buf[pl.ds(j * 16, 16)] * invv
        return carry

    lax.fori_loop(0, nrows, row_body, 0)


def _sc_weights_body(l4x, l3x, l2x, wT1, wT2,
                     sx1, sy1, sz1, dx1, dy1, dz1, d2b1, ob1,
                     sx2, sy2, sz2, dx2, dy2, dz2, d2b2, ob2):
    wid = lax.axis_index("s") * 2 + lax.axis_index("c")
    b = wid // 4
    q = wid % 4

    pltpu.sync_copy(l4x.at[pl.ds((b * 3 + 0) * 64, 64)], sx1)
    pltpu.sync_copy(l4x.at[pl.ds((b * 3 + 1) * 64, 64)], sy1)
    pltpu.sync_copy(l4x.at[pl.ds((b * 3 + 2) * 64, 64)], sz1)
    pltpu.sync_copy(l3x.at[pl.ds((b * 3 + 0) * 256 + q * 64, 64)], dx1)
    pltpu.sync_copy(l3x.at[pl.ds((b * 3 + 1) * 256 + q * 64, 64)], dy1)
    pltpu.sync_copy(l3x.at[pl.ds((b * 3 + 2) * 256 + q * 64, 64)], dz1)
    _sc_stage_rows(64, 4, 3.125, sx1, sy1, sz1, dx1, dy1, dz1, d2b1, ob1)
    pltpu.sync_copy(ob1, wT1.at[pl.ds((b * 256 + q * 64) * 64, 64 * 64)])

    pltpu.sync_copy(l3x.at[pl.ds((b * 3 + 0) * 256, 256)], sx2)
    pltpu.sync_copy(l3x.at[pl.ds((b * 3 + 1) * 256, 256)], sy2)
    pltpu.sync_copy(l3x.at[pl.ds((b * 3 + 2) * 256, 256)], sz2)
    pltpu.sync_copy(l2x.at[pl.ds((b * 3 + 0) * 1024 + q * 256, 256)], dx2)
    pltpu.sync_copy(l2x.at[pl.ds((b * 3 + 1) * 1024 + q * 256, 256)], dy2)
    pltpu.sync_copy(l2x.at[pl.ds((b * 3 + 2) * 1024 + q * 256, 256)], dz2)
    _sc_stage_rows(256, 16, 12.5, sx2, sy2, sz2, dx2, dy2, dz2, d2b2, ob2)
    pltpu.sync_copy(ob2, wT2.at[pl.ds((b * 1024 + q * 256) * 256, 256 * 256)])


def _sc_weights(l4_xyz, l3_xyz, l2_xyz):
    B = l4_xyz.shape[0]
    mesh = plsc.VectorSubcoreMesh(core_axis_name="c", subcore_axis_name="s",
                                  num_cores=2, num_subcores=16)
    f32 = jnp.float32
    run = pl.kernel(
        _sc_weights_body,
        out_type=(jax.ShapeDtypeStruct((B * 256 * 64,), f32),
                  jax.ShapeDtypeStruct((B * 1024 * 256,), f32)),
        mesh=mesh,
        compiler_params=pltpu.CompilerParams(needs_layout_passes=False),
        scratch_types=[
            pltpu.VMEM((64,), f32), pltpu.VMEM((64,), f32), pltpu.VMEM((64,), f32),
            pltpu.VMEM((64,), f32), pltpu.VMEM((64,), f32), pltpu.VMEM((64,), f32),
            pltpu.VMEM((64,), f32), pltpu.VMEM((64 * 64,), f32),
            pltpu.VMEM((256,), f32), pltpu.VMEM((256,), f32), pltpu.VMEM((256,), f32),
            pltpu.VMEM((256,), f32), pltpu.VMEM((256,), f32), pltpu.VMEM((256,), f32),
            pltpu.VMEM((256,), f32), pltpu.VMEM((256 * 256,), f32),
        ],
    )
    wT1f, wT2f = run(l4_xyz.reshape(-1), l3_xyz.reshape(-1), l2_xyz.reshape(-1))
    return wT1f.reshape(B, 256, 64), wT2f.reshape(B, 1024, 256)


def _tc_stage(wT, spoints, dpoints, Wa, ba, Wb, bb):
    interp = lax.dot_general(spoints, wT, (((1,), (1,)), ((), ())),
                             preferred_element_type=jnp.float32)
    new = interp + dpoints
    h = jnp.maximum(jnp.dot(Wa, new, preferred_element_type=jnp.float32) + ba, 0.0)
    return jnp.maximum(jnp.dot(Wb, h, preferred_element_type=jnp.float32) + bb, 0.0)


def _tc_body(wT1_ref, l4p_ref, l3p_ref, wT2_ref, l2p_ref,
             W1_ref, b1_ref, W2_ref, b2_ref, W3_ref, b3_ref, W4_ref, b4_ref,
             out_ref):
    l3_new = _tc_stage(wT1_ref[0], l4p_ref[0], l3p_ref[0],
                       W1_ref[...], b1_ref[...], W2_ref[...], b2_ref[...])
    out_ref[0] = _tc_stage(wT2_ref[0], l3_new, l2p_ref[0],
                           W3_ref[...], b3_ref[...], W4_ref[...], b4_ref[...])


def kernel(l1_xyz, l1_points, l2_xyz, l2_points, l3_xyz, l3_points, l4_xyz,
           l4_points, W1, b1, W2, b2, W3, b3, W4, b4):
    del l1_xyz, l1_points
    B = l2_xyz.shape[0]
    wT1, wT2 = _sc_weights(l4_xyz, l3_xyz, l2_xyz)
    b1c, b2c = b1[:, None], b2[:, None]
    b3c, b4c = b3[:, None], b4[:, None]

    def bspec(shape):
        return pl.BlockSpec((1,) + shape, lambda b: (b, 0, 0))

    def wspec(shape):
        return pl.BlockSpec(shape, lambda b: (0,) * len(shape))

    return pl.pallas_call(
        _tc_body,
        grid=(B,),
        in_specs=[
            bspec((256, 64)), bspec((512, 64)), bspec((512, 256)),
            bspec((1024, 256)), bspec((512, 1024)),
            wspec((512, 512)), wspec((512, 1)), wspec((512, 512)), wspec((512, 1)),
            wspec((256, 512)), wspec((256, 1)), wspec((256, 256)), wspec((256, 1)),
        ],
        out_specs=pl.BlockSpec((1, 256, 1024), lambda b: (b, 0, 0)),
        out_shape=jax.ShapeDtypeStruct((B, 256, 1024), jnp.float32),
    )(wT1, l4_points, l3_points, wT2, l2_points,
      W1, b1c, W2, b2c, W3, b3c, W4, b4c)


# speedup vs baseline: 1.3288x; 1.3288x over previous
import functools

import jax
import jax.numpy as jnp
from jax import lax
from jax.experimental import pallas as pl
from jax.experimental.pallas import tpu as pltpu
from jax.experimental.pallas import tpu_sc as plsc


def _top16_sorted(d2_vregs):
    runs = [lax.sort(v) for v in d2_vregs]
    while len(runs) > 1:
        nxt = []
        for a, b in zip(runs[0::2], runs[1::2]):
            nxt.append(lax.sort(jnp.minimum(a, lax.rev(b, (0,)))))
        runs = nxt
    return runs[0]


_GDN = lax.GatherDimensionNumbers(offset_dims=(), collapsed_slice_dims=(0,),
                                  start_index_map=(0,))


def _lane_perm(v, idx16):
    return lax.gather(v, idx16[:, None], _GDN, slice_sizes=(1,),
                      mode=lax.GatherScatterMode.PROMISE_IN_BOUNDS)


def _lane_sum(v):
    lane = lax.iota(jnp.int32, 16)
    for sh in (8, 4, 2, 1):
        v = v + _lane_perm(v, (lane + sh) % 16)
    return v


def _sc_stage_rows(nrows, nsv, inv2bw2, sxv, syv, szv, dxv, dyv, dzv, obuf):
    ns = nsv * 16

    @plsc.parallel_loop(0, nrows, step=1, unroll=2)
    def row_body(r):
        cb = (r // 16) * 16
        off16 = jnp.full((16,), r - cb, jnp.int32)
        dnx = _lane_perm(dxv[pl.ds(cb, 16)], off16)
        dny = _lane_perm(dyv[pl.ds(cb, 16)], off16)
        dnz = _lane_perm(dzv[pl.ds(cb, 16)], off16)
        d2s = []
        for j in range(nsv):
            sl = pl.ds(j * 16, 16)
            ax = sxv[sl] - dnx
            ay = syv[sl] - dny
            az = szv[sl] - dnz
            d2s.append(ax * ax + ay * ay + az * az)
        t16 = _top16_sorted(d2s)
        tv = _lane_perm(t16, jnp.full((16,), 15, jnp.int32))
        ws = [jnp.where(d2 <= tv, jnp.exp(d2 * (-inv2bw2)), 0.0) for d2 in d2s]
        accs = ws
        while len(accs) > 1:
            accs = [a + b for a, b in zip(accs[0::2], accs[1::2])]
        denom = _lane_sum(accs[0]) + 1e-8
        invv = jnp.ones((16,), jnp.float32) / denom
        for j in range(nsv):
            obuf[pl.ds(r * ns + j * 16, 16)] = ws[j] * invv


def _sc_weights_body(l4x, l3x, l2x, wT1, wT2,
                     sx1, sy1, sz1, dx1, dy1, dz1, ob1,
                     sx2, sy2, sz2, dx2, dy2, dz2, ob2):
    wid = lax.axis_index("s") * 2 + lax.axis_index("c")
    b = wid // 4
    q = wid % 4

    pltpu.sync_copy(l4x.at[pl.ds((b * 3 + 0) * 64, 64)], sx1)
    pltpu.sync_copy(l4x.at[pl.ds((b * 3 + 1) * 64, 64)], sy1)
    pltpu.sync_copy(l4x.at[pl.ds((b * 3 + 2) * 64, 64)], sz1)
    pltpu.sync_copy(l3x.at[pl.ds((b * 3 + 0) * 256 + q * 64, 64)], dx1)
    pltpu.sync_copy(l3x.at[pl.ds((b * 3 + 1) * 256 + q * 64, 64)], dy1)
    pltpu.sync_copy(l3x.at[pl.ds((b * 3 + 2) * 256 + q * 64, 64)], dz1)
    _sc_stage_rows(64, 4, 3.125, sx1, sy1, sz1, dx1, dy1, dz1, ob1)
    pltpu.sync_copy(ob1, wT1.at[pl.ds((b * 256 + q * 64) * 64, 64 * 64)])

    pltpu.sync_copy(l3x.at[pl.ds((b * 3 + 0) * 256, 256)], sx2)
    pltpu.sync_copy(l3x.at[pl.ds((b * 3 + 1) * 256, 256)], sy2)
    pltpu.sync_copy(l3x.at[pl.ds((b * 3 + 2) * 256, 256)], sz2)
    pltpu.sync_copy(l2x.at[pl.ds((b * 3 + 0) * 1024 + q * 256, 256)], dx2)
    pltpu.sync_copy(l2x.at[pl.ds((b * 3 + 1) * 1024 + q * 256, 256)], dy2)
    pltpu.sync_copy(l2x.at[pl.ds((b * 3 + 2) * 1024 + q * 256, 256)], dz2)
    _sc_stage_rows(256, 16, 12.5, sx2, sy2, sz2, dx2, dy2, dz2, ob2)
    pltpu.sync_copy(ob2, wT2.at[pl.ds((b * 1024 + q * 256) * 256, 256 * 256)])


def _sc_weights(l4_xyz, l3_xyz, l2_xyz):
    B = l4_xyz.shape[0]
    mesh = plsc.VectorSubcoreMesh(core_axis_name="c", subcore_axis_name="s",
                                  num_cores=2, num_subcores=16)
    f32 = jnp.float32
    run = pl.kernel(
        _sc_weights_body,
        out_type=(jax.ShapeDtypeStruct((B * 256 * 64,), f32),
                  jax.ShapeDtypeStruct((B * 1024 * 256,), f32)),
        mesh=mesh,
        compiler_params=pltpu.CompilerParams(needs_layout_passes=False),
        scratch_types=[
            pltpu.VMEM((64,), f32), pltpu.VMEM((64,), f32), pltpu.VMEM((64,), f32),
            pltpu.VMEM((64,), f32), pltpu.VMEM((64,), f32), pltpu.VMEM((64,), f32),
            pltpu.VMEM((64 * 64,), f32),
            pltpu.VMEM((256,), f32), pltpu.VMEM((256,), f32), pltpu.VMEM((256,), f32),
            pltpu.VMEM((256,), f32), pltpu.VMEM((256,), f32), pltpu.VMEM((256,), f32),
            pltpu.VMEM((256 * 256,), f32),
        ],
    )
    wT1f, wT2f = run(l4_xyz.reshape(-1), l3_xyz.reshape(-1), l2_xyz.reshape(-1))
    return wT1f.reshape(B, 256, 64), wT2f.reshape(B, 1024, 256)


def _tc_stage(wT, spoints, dpoints, Wa, ba, Wb, bb):
    interp = lax.dot_general(spoints, wT, (((1,), (1,)), ((), ())),
                             preferred_element_type=jnp.float32)
    new = interp + dpoints
    h = jnp.maximum(jnp.dot(Wa, new, preferred_element_type=jnp.float32) + ba, 0.0)
    return jnp.maximum(jnp.dot(Wb, h, preferred_element_type=jnp.float32) + bb, 0.0)


def _tc_body(wT1_ref, l4p_ref, l3p_ref, wT2_ref, l2p_ref,
             W1_ref, b1_ref, W2_ref, b2_ref, W3_ref, b3_ref, W4_ref, b4_ref,
             out_ref):
    l3_new = _tc_stage(wT1_ref[0], l4p_ref[0], l3p_ref[0],
                       W1_ref[...], b1_ref[...], W2_ref[...], b2_ref[...])
    out_ref[0] = _tc_stage(wT2_ref[0], l3_new, l2p_ref[0],
                           W3_ref[...], b3_ref[...], W4_ref[...], b4_ref[...])


def kernel(l1_xyz, l1_points, l2_xyz, l2_points, l3_xyz, l3_points, l4_xyz,
           l4_points, W1, b1, W2, b2, W3, b3, W4, b4):
    del l1_xyz, l1_points
    B = l2_xyz.shape[0]
    wT1, wT2 = _sc_weights(l4_xyz, l3_xyz, l2_xyz)
    b1c, b2c = b1[:, None], b2[:, None]
    b3c, b4c = b3[:, None], b4[:, None]

    def bspec(shape):
        return pl.BlockSpec((1,) + shape, lambda b: (b, 0, 0))

    def wspec(shape):
        return pl.BlockSpec(shape, lambda b: (0,) * len(shape))

    return pl.pallas_call(
        _tc_body,
        grid=(B,),
        in_specs=[
            bspec((256, 64)), bspec((512, 64)), bspec((512, 256)),
            bspec((1024, 256)), bspec((512, 1024)),
            wspec((512, 512)), wspec((512, 1)), wspec((512, 512)), wspec((512, 1)),
            wspec((256, 512)), wspec((256, 1)), wspec((256, 256)), wspec((256, 1)),
        ],
        out_specs=pl.BlockSpec((1, 256, 1024), lambda b: (b, 0, 0)),
        out_shape=jax.ShapeDtypeStruct((B, 256, 1024), jnp.float32),
    )(wT1, l4_points, l3_points, wT2, l2_points,
      W1, b1c, W2, b2c, W3, b3c, W4, b4c)


# speedup vs baseline: 1.3798x; 1.0384x over previous
import functools

import jax
import jax.numpy as jnp
from jax import lax
from jax.experimental import pallas as pl
from jax.experimental.pallas import tpu as pltpu
from jax.experimental.pallas import tpu_sc as plsc


def _top16_sorted(d2_vregs):
    runs = [lax.sort(v) for v in d2_vregs]
    while len(runs) > 1:
        nxt = []
        for a, b in zip(runs[0::2], runs[1::2]):
            nxt.append(lax.sort(jnp.minimum(a, lax.rev(b, (0,)))))
        runs = nxt
    return runs[0]


_GDN = lax.GatherDimensionNumbers(offset_dims=(), collapsed_slice_dims=(0,),
                                  start_index_map=(0,))


def _lane_perm(v, idx16):
    return lax.gather(v, idx16[:, None], _GDN, slice_sizes=(1,),
                      mode=lax.GatherScatterMode.PROMISE_IN_BOUNDS)


def _lane_sum(v):
    lane = lax.iota(jnp.int32, 16)
    for sh in (8, 4, 2, 1):
        v = v + _lane_perm(v, (lane + sh) % 16)
    return v


def _sc_stage_rows(nrows, nsv, inv2bw2, sxv, syv, szv, dxv, dyv, dzv, obuf):
    ns = nsv * 16

    @plsc.parallel_loop(0, nrows, step=1, unroll=4)
    def row_body(r):
        cb = (r // 16) * 16
        off16 = jnp.full((16,), r - cb, jnp.int32)
        dnx = _lane_perm(dxv[pl.ds(cb, 16)], off16)
        dny = _lane_perm(dyv[pl.ds(cb, 16)], off16)
        dnz = _lane_perm(dzv[pl.ds(cb, 16)], off16)
        d2s = []
        for j in range(nsv):
            sl = pl.ds(j * 16, 16)
            ax = sxv[sl] - dnx
            ay = syv[sl] - dny
            az = szv[sl] - dnz
            d2s.append(ax * ax + ay * ay + az * az)
        t16 = _top16_sorted(d2s)
        tv = _lane_perm(t16, jnp.full((16,), 15, jnp.int32))
        ws = [jnp.where(d2 <= tv, jnp.exp(d2 * (-inv2bw2)), 0.0) for d2 in d2s]
        accs = ws
        while len(accs) > 1:
            accs = [a + b for a, b in zip(accs[0::2], accs[1::2])]
        denom = _lane_sum(accs[0]) + 1e-8
        invv = jnp.ones((16,), jnp.float32) / denom
        for j in range(nsv):
            obuf[pl.ds(r * ns + j * 16, 16)] = ws[j] * invv


def _sc_weights_body(l4x, l3x, l2x, wT1, wT2,
                     sx1, sy1, sz1, dx1, dy1, dz1, ob1,
                     sx2, sy2, sz2, dx2, dy2, dz2, ob2):
    wid = lax.axis_index("s") * 2 + lax.axis_index("c")
    b = wid // 4
    q = wid % 4

    pltpu.sync_copy(l4x.at[pl.ds((b * 3 + 0) * 64, 64)], sx1)
    pltpu.sync_copy(l4x.at[pl.ds((b * 3 + 1) * 64, 64)], sy1)
    pltpu.sync_copy(l4x.at[pl.ds((b * 3 + 2) * 64, 64)], sz1)
    pltpu.sync_copy(l3x.at[pl.ds((b * 3 + 0) * 256 + q * 64, 64)], dx1)
    pltpu.sync_copy(l3x.at[pl.ds((b * 3 + 1) * 256 + q * 64, 64)], dy1)
    pltpu.sync_copy(l3x.at[pl.ds((b * 3 + 2) * 256 + q * 64, 64)], dz1)
    _sc_stage_rows(64, 4, 3.125, sx1, sy1, sz1, dx1, dy1, dz1, ob1)
    pltpu.sync_copy(ob1, wT1.at[pl.ds((b * 256 + q * 64) * 64, 64 * 64)])

    pltpu.sync_copy(l3x.at[pl.ds((b * 3 + 0) * 256, 256)], sx2)
    pltpu.sync_copy(l3x.at[pl.ds((b * 3 + 1) * 256, 256)], sy2)
    pltpu.sync_copy(l3x.at[pl.ds((b * 3 + 2) * 256, 256)], sz2)
    pltpu.sync_copy(l2x.at[pl.ds((b * 3 + 0) * 1024 + q * 256, 256)], dx2)
    pltpu.sync_copy(l2x.at[pl.ds((b * 3 + 1) * 1024 + q * 256, 256)], dy2)
    pltpu.sync_copy(l2x.at[pl.ds((b * 3 + 2) * 1024 + q * 256, 256)], dz2)
    _sc_stage_rows(256, 16, 12.5, sx2, sy2, sz2, dx2, dy2, dz2, ob2)
    pltpu.sync_copy(ob2, wT2.at[pl.ds((b * 1024 + q * 256) * 256, 256 * 256)])


def _sc_weights(l4_xyz, l3_xyz, l2_xyz):
    B = l4_xyz.shape[0]
    mesh = plsc.VectorSubcoreMesh(core_axis_name="c", subcore_axis_name="s",
                                  num_cores=2, num_subcores=16)
    f32 = jnp.float32
    run = pl.kernel(
        _sc_weights_body,
        out_type=(jax.ShapeDtypeStruct((B * 256 * 64,), f32),
                  jax.ShapeDtypeStruct((B * 1024 * 256,), f32)),
        mesh=mesh,
        compiler_params=pltpu.CompilerParams(needs_layout_passes=False),
        scratch_types=[
            pltpu.VMEM((64,), f32), pltpu.VMEM((64,), f32), pltpu.VMEM((64,), f32),
            pltpu.VMEM((64,), f32), pltpu.VMEM((64,), f32), pltpu.VMEM((64,), f32),
            pltpu.VMEM((64 * 64,), f32),
            pltpu.VMEM((256,), f32), pltpu.VMEM((256,), f32), pltpu.VMEM((256,), f32),
            pltpu.VMEM((256,), f32), pltpu.VMEM((256,), f32), pltpu.VMEM((256,), f32),
            pltpu.VMEM((256 * 256,), f32),
        ],
    )
    wT1f, wT2f = run(l4_xyz.reshape(-1), l3_xyz.reshape(-1), l2_xyz.reshape(-1))
    return wT1f.reshape(B, 256, 64), wT2f.reshape(B, 1024, 256)


def _tc_stage(wT, spoints, dpoints, Wa, ba, Wb, bb):
    interp = lax.dot_general(spoints, wT, (((1,), (1,)), ((), ())),
                             preferred_element_type=jnp.float32)
    new = interp + dpoints
    h = jnp.maximum(jnp.dot(Wa, new, preferred_element_type=jnp.float32) + ba, 0.0)
    return jnp.maximum(jnp.dot(Wb, h, preferred_element_type=jnp.float32) + bb, 0.0)


def _tc_body(wT1_ref, l4p_ref, l3p_ref, wT2_ref, l2p_ref,
             W1_ref, b1_ref, W2_ref, b2_ref, W3_ref, b3_ref, W4_ref, b4_ref,
             out_ref):
    l3_new = _tc_stage(wT1_ref[0], l4p_ref[0], l3p_ref[0],
                       W1_ref[...], b1_ref[...], W2_ref[...], b2_ref[...])
    out_ref[0] = _tc_stage(wT2_ref[0], l3_new, l2p_ref[0],
                           W3_ref[...], b3_ref[...], W4_ref[...], b4_ref[...])


def kernel(l1_xyz, l1_points, l2_xyz, l2_points, l3_xyz, l3_points, l4_xyz,
           l4_points, W1, b1, W2, b2, W3, b3, W4, b4):
    del l1_xyz, l1_points
    B = l2_xyz.shape[0]
    wT1, wT2 = _sc_weights(l4_xyz, l3_xyz, l2_xyz)
    b1c, b2c = b1[:, None], b2[:, None]
    b3c, b4c = b3[:, None], b4[:, None]

    def bspec(shape):
        return pl.BlockSpec((1,) + shape, lambda b: (b, 0, 0))

    def wspec(shape):
        return pl.BlockSpec(shape, lambda b: (0,) * len(shape))

    return pl.pallas_call(
        _tc_body,
        grid=(B,),
        in_specs=[
            bspec((256, 64)), bspec((512, 64)), bspec((512, 256)),
            bspec((1024, 256)), bspec((512, 1024)),
            wspec((512, 512)), wspec((512, 1)), wspec((512, 512)), wspec((512, 1)),
            wspec((256, 512)), wspec((256, 1)), wspec((256, 256)), wspec((256, 1)),
        ],
        out_specs=pl.BlockSpec((1, 256, 1024), lambda b: (b, 0, 0)),
        out_shape=jax.ShapeDtypeStruct((B, 256, 1024), jnp.float32),
    )(wT1, l4_points, l3_points, wT2, l2_points,
      W1, b1c, W2, b2c, W3, b3c, W4, b4c)
